# baseline (speedup 1.0000x reference)
import jax
import jax.numpy as jnp
from jax.experimental import pallas as pl
from jax.experimental.pallas import tpu as pltpu

Ns, Ng, D = 8192, 4096, 256
TI = 512
NI = Ns // TI
TJ = 1024
NJ = Ng // TJ
NC = 4
KC = Ng // NC

_F32 = jnp.float32
_BF16 = jnp.bfloat16


def _dot(a, b):
    return jnp.dot(a, b, preferred_element_type=_F32)


def _body(*refs):
    chunks = refs[:NC]
    (sx_ref, gx_ref,
     wsp_ref, bsp_ref, wgp_ref, bgp_ref,
     wsa_ref, wsb_ref, bspo_ref, wga_ref, wgb_ref, bgpo_ref,
     xs_ref, xg_ref, xgp_ref, accT_ref) = refs[NC:]
    i = pl.program_id(0)
    first = i == 0

    @pl.when(first)
    def _():
        g = jnp.maximum(_dot(gx_ref[...].astype(_BF16), wgp_ref[...])
                        + bgp_ref[...], 0.0)
        xgp_ref[...] = g.astype(_BF16)

    s = jnp.maximum(_dot(sx_ref[...].astype(_BF16), wsp_ref[...])
                    + bsp_ref[...], 0.0)
    s16 = s.astype(_BF16)
    sT = s16.T

    a = [r_ref[...].astype(_BF16) for r_ref in chunks]

    for c in range(NC):
        contrib = _dot(sT, a[c])
        sl = pl.ds(c * KC, KC)
        accT_ref[:, sl] = jnp.where(first, contrib, accT_ref[:, sl] + contrib)

    parts = [_dot(a[c], xgp_ref[pl.ds(c * KC, KC), :]) for c in range(NC)]
    while len(parts) > 1:
        parts = [parts[k] + parts[k + 1] for k in range(0, len(parts), 2)]
    msg = parts[0]
    r = (_dot(s16, wsa_ref[...])
         + _dot(msg.astype(_BF16), wsb_ref[...])
         + bspo_ref[...])
    xs_ref[...] = jnp.maximum(r, 0.0)

    @pl.when(i == NI - 1)
    def _():
        for t in range(NJ):
            sl = pl.ds(t * TJ, TJ)
            acc = accT_ref[:, sl].astype(_BF16).T
            rg = (_dot(xgp_ref[sl, :], wga_ref[...])
                  + _dot(acc, wgb_ref[...])
                  + bgpo_ref[...])
            xg_ref[sl, :] = jnp.maximum(rg, 0.0)


@jax.jit
def _run(sx, gx, rbf, wsp, bsp, wgp, bgp, wsa, wsb, bspo, wga, wgb, bgpo):
    xs, xg = pl.pallas_call(
        _body,
        grid=(NI,),
        in_specs=[
            *[pl.BlockSpec((TI, KC), lambda i, c=c: (i, c)) for c in range(NC)],
            pl.BlockSpec((TI, D), lambda i: (i, 0)),
            pl.BlockSpec((Ng, D), lambda i: (0, 0)),
            pl.BlockSpec((D, D), lambda i: (0, 0)),
            pl.BlockSpec((1, D), lambda i: (0, 0)),
            pl.BlockSpec((D, D), lambda i: (0, 0)),
            pl.BlockSpec((1, D), lambda i: (0, 0)),
            pl.BlockSpec((D, D), lambda i: (0, 0)),
            pl.BlockSpec((D, D), lambda i: (0, 0)),
            pl.BlockSpec((1, D), lambda i: (0, 0)),
            pl.BlockSpec((D, D), lambda i: (0, 0)),
            pl.BlockSpec((D, D), lambda i: (0, 0)),
            pl.BlockSpec((1, D), lambda i: (0, 0)),
        ],
        out_specs=[
            pl.BlockSpec((TI, D), lambda i: (i, 0)),
            pl.BlockSpec((Ng, D), lambda i: (0, 0)),
        ],
        out_shape=[
            jax.ShapeDtypeStruct((Ns, D), _F32),
            jax.ShapeDtypeStruct((Ng, D), _F32),
        ],
        scratch_shapes=[
            pltpu.VMEM((Ng, D), _BF16),
            pltpu.VMEM((D, Ng), _F32),
        ],
        compiler_params=pltpu.CompilerParams(
            dimension_semantics=("arbitrary",),
            vmem_limit_bytes=64 * 1024 * 1024,
        ),
    )(*([rbf] * NC), sx, gx,
      wsp, bsp, wgp, bgp, wsa, wsb, bspo, wga, wgb, bgpo)
    return xs, xg


def kernel(surface_x, graph_x, rbf_weights,
           W_s_pre, b_s_pre, W_g_pre, b_g_pre,
           W_s_post, b_s_post, W_g_post, b_g_post):
    wsp = W_s_pre.astype(_BF16)
    wgp = W_g_pre.astype(_BF16)
    wsa = W_s_post[:D].astype(_BF16)
    wsb = W_s_post[D:].astype(_BF16)
    wga = W_g_post[:D].astype(_BF16)
    wgb = W_g_post[D:].astype(_BF16)
    bsp = b_s_pre.reshape(1, D)
    bgp = b_g_pre.reshape(1, D)
    bspo = b_s_post.reshape(1, D)
    bgpo = b_g_post.reshape(1, D)
    xs, xg = _run(surface_x, graph_x, rbf_weights, wsp, bsp, wgp, bgp,
                  wsa, wsb, bspo, wga, wgb, bgpo)
    return (xs, xg)


# speedup vs baseline: 1.0956x; 1.0956x over previous
import jax
import jax.numpy as jnp
from jax.experimental import pallas as pl
from jax.experimental.pallas import tpu as pltpu

Ns, Ng, D = 8192, 4096, 256
TI = 1024
NI = Ns // TI
TJ = 1024
NJ = Ng // TJ
NC = 1
KC = Ng // NC

_F32 = jnp.float32
_BF16 = jnp.bfloat16


def _dot(a, b):
    return jnp.dot(a, b, preferred_element_type=_F32)


def _body(*refs):
    chunks = refs[:NC]
    (sx_ref, gx_ref,
     wsp_ref, bsp_ref, wgp_ref, bgp_ref,
     wsa_ref, wsb_ref, bspo_ref, wga_ref, wgb_ref, bgpo_ref,
     xs_ref, xg_ref, xgp_ref, accT_ref) = refs[NC:]
    i = pl.program_id(0)
    first = i == 0

    @pl.when(first)
    def _():
        g = jnp.maximum(_dot(gx_ref[...].astype(_BF16), wgp_ref[...])
                        + bgp_ref[...], 0.0)
        xgp_ref[...] = g.astype(_BF16)

    s = jnp.maximum(_dot(sx_ref[...].astype(_BF16), wsp_ref[...])
                    + bsp_ref[...], 0.0)
    s16 = s.astype(_BF16)
    sT = s16.T

    a = [r_ref[...].astype(_BF16) for r_ref in chunks]

    for c in range(NC):
        contrib = _dot(sT, a[c])
        sl = pl.ds(c * KC, KC)
        accT_ref[:, sl] = jnp.where(first, contrib, accT_ref[:, sl] + contrib)

    parts = [_dot(a[c], xgp_ref[pl.ds(c * KC, KC), :]) for c in range(NC)]
    while len(parts) > 1:
        parts = [parts[k] + parts[k + 1] for k in range(0, len(parts), 2)]
    msg = parts[0]
    r = (_dot(s16, wsa_ref[...])
         + _dot(msg.astype(_BF16), wsb_ref[...])
         + bspo_ref[...])
    xs_ref[...] = jnp.maximum(r, 0.0)

    @pl.when(i == NI - 1)
    def _():
        for t in range(NJ):
            sl = pl.ds(t * TJ, TJ)
            acc = accT_ref[:, sl].astype(_BF16).T
            rg = (_dot(xgp_ref[sl, :], wga_ref[...])
                  + _dot(acc, wgb_ref[...])
                  + bgpo_ref[...])
            xg_ref[sl, :] = jnp.maximum(rg, 0.0)


@jax.jit
def _run(sx, gx, rbf, wsp, bsp, wgp, bgp, wsa, wsb, bspo, wga, wgb, bgpo):
    xs, xg = pl.pallas_call(
        _body,
        grid=(NI,),
        in_specs=[
            *[pl.BlockSpec((TI, KC), lambda i, c=c: (i, c)) for c in range(NC)],
            pl.BlockSpec((TI, D), lambda i: (i, 0)),
            pl.BlockSpec((Ng, D), lambda i: (0, 0)),
            pl.BlockSpec((D, D), lambda i: (0, 0)),
            pl.BlockSpec((1, D), lambda i: (0, 0)),
            pl.BlockSpec((D, D), lambda i: (0, 0)),
            pl.BlockSpec((1, D), lambda i: (0, 0)),
            pl.BlockSpec((D, D), lambda i: (0, 0)),
            pl.BlockSpec((D, D), lambda i: (0, 0)),
            pl.BlockSpec((1, D), lambda i: (0, 0)),
            pl.BlockSpec((D, D), lambda i: (0, 0)),
            pl.BlockSpec((D, D), lambda i: (0, 0)),
            pl.BlockSpec((1, D), lambda i: (0, 0)),
        ],
        out_specs=[
            pl.BlockSpec((TI, D), lambda i: (i, 0)),
            pl.BlockSpec((Ng, D), lambda i: (0, 0)),
        ],
        out_shape=[
            jax.ShapeDtypeStruct((Ns, D), _F32),
            jax.ShapeDtypeStruct((Ng, D), _F32),
        ],
        scratch_shapes=[
            pltpu.VMEM((Ng, D), _BF16),
            pltpu.VMEM((D, Ng), _F32),
        ],
        compiler_params=pltpu.CompilerParams(
            dimension_semantics=("arbitrary",),
            vmem_limit_bytes=64 * 1024 * 1024,
        ),
    )(*([rbf] * NC), sx, gx,
      wsp, bsp, wgp, bgp, wsa, wsb, bspo, wga, wgb, bgpo)
    return xs, xg


def kernel(surface_x, graph_x, rbf_weights,
           W_s_pre, b_s_pre, W_g_pre, b_g_pre,
           W_s_post, b_s_post, W_g_post, b_g_post):
    wsp = W_s_pre.astype(_BF16)
    wgp = W_g_pre.astype(_BF16)
    wsa = W_s_post[:D].astype(_BF16)
    wsb = W_s_post[D:].astype(_BF16)
    wga = W_g_post[:D].astype(_BF16)
    wgb = W_g_post[D:].astype(_BF16)
    bsp = b_s_pre.reshape(1, D)
    bgp = b_g_pre.reshape(1, D)
    bspo = b_s_post.reshape(1, D)
    bgpo = b_g_post.reshape(1, D)
    xs, xg = _run(surface_x, graph_x, rbf_weights, wsp, bsp, wgp, bgp,
                  wsa, wsb, bspo, wga, wgb, bgpo)
    return (xs, xg)


# speedup vs baseline: 1.0978x; 1.0020x over previous
import jax
import jax.numpy as jnp
from jax.experimental import pallas as pl
from jax.experimental.pallas import tpu as pltpu

Ns, Ng, D = 8192, 4096, 256
TI = 1024
NI = Ns // TI
TJ = 1024
NJ = Ng // TJ
NC = 1
KC = Ng // NC

_F32 = jnp.float32
_BF16 = jnp.bfloat16


def _dot(a, b):
    return jnp.dot(a, b, preferred_element_type=_F32)


def _body(*refs):
    chunks = refs[:NC]
    (sx_ref, gx_ref,
     wsp_ref, bsp_ref, wgp_ref, bgp_ref,
     wsa_ref, wsb_ref, bspo_ref, wga_ref, wgb_ref, bgpo_ref,
     xs_ref, xg_ref, xgp_ref, accT_ref, xgpa_ref) = refs[NC:]
    i = pl.program_id(0)
    first = i == 0

    @pl.when(first)
    def _():
        g = jnp.maximum(_dot(gx_ref[...].astype(_BF16), wgp_ref[...])
                        + bgp_ref[...], 0.0)
        g16 = g.astype(_BF16)
        xgp_ref[...] = g16
        xgpa_ref[...] = _dot(g16, wga_ref[...]) + bgpo_ref[...]

    s = jnp.maximum(_dot(sx_ref[...].astype(_BF16), wsp_ref[...])
                    + bsp_ref[...], 0.0)
    s16 = s.astype(_BF16)
    sT = s16.T

    a = [r_ref[...].astype(_BF16) for r_ref in chunks]

    for c in range(NC):
        contrib = _dot(sT, a[c])
        sl = pl.ds(c * KC, KC)
        accT_ref[:, sl] = jnp.where(first, contrib, accT_ref[:, sl] + contrib)

    parts = [_dot(a[c], xgp_ref[pl.ds(c * KC, KC), :]) for c in range(NC)]
    while len(parts) > 1:
        parts = [parts[k] + parts[k + 1] for k in range(0, len(parts), 2)]
    msg = parts[0]
    r = (_dot(s16, wsa_ref[...])
         + _dot(msg.astype(_BF16), wsb_ref[...])
         + bspo_ref[...])
    xs_ref[...] = jnp.maximum(r, 0.0)

    @pl.when(i == NI - 1)
    def _():
        for t in range(NJ):
            sl = pl.ds(t * TJ, TJ)
            acc = accT_ref[:, sl].astype(_BF16).T
            rg = xgpa_ref[sl, :] + _dot(acc, wgb_ref[...])
            xg_ref[sl, :] = jnp.maximum(rg, 0.0)


@jax.jit
def _run(sx, gx, rbf, wsp, bsp, wgp, bgp, wsa, wsb, bspo, wga, wgb, bgpo):
    xs, xg = pl.pallas_call(
        _body,
        grid=(NI,),
        in_specs=[
            *[pl.BlockSpec((TI, KC), lambda i, c=c: (i, c)) for c in range(NC)],
            pl.BlockSpec((TI, D), lambda i: (i, 0)),
            pl.BlockSpec((Ng, D), lambda i: (0, 0)),
            pl.BlockSpec((D, D), lambda i: (0, 0)),
            pl.BlockSpec((1, D), lambda i: (0, 0)),
            pl.BlockSpec((D, D), lambda i: (0, 0)),
            pl.BlockSpec((1, D), lambda i: (0, 0)),
            pl.BlockSpec((D, D), lambda i: (0, 0)),
            pl.BlockSpec((D, D), lambda i: (0, 0)),
            pl.BlockSpec((1, D), lambda i: (0, 0)),
            pl.BlockSpec((D, D), lambda i: (0, 0)),
            pl.BlockSpec((D, D), lambda i: (0, 0)),
            pl.BlockSpec((1, D), lambda i: (0, 0)),
        ],
        out_specs=[
            pl.BlockSpec((TI, D), lambda i: (i, 0)),
            pl.BlockSpec((Ng, D), lambda i: (0, 0)),
        ],
        out_shape=[
            jax.ShapeDtypeStruct((Ns, D), _F32),
            jax.ShapeDtypeStruct((Ng, D), _F32),
        ],
        scratch_shapes=[
            pltpu.VMEM((Ng, D), _BF16),
            pltpu.VMEM((D, Ng), _F32),
            pltpu.VMEM((Ng, D), _F32),
        ],
        compiler_params=pltpu.CompilerParams(
            dimension_semantics=("arbitrary",),
            vmem_limit_bytes=64 * 1024 * 1024,
        ),
    )(*([rbf] * NC), sx, gx,
      wsp, bsp, wgp, bgp, wsa, wsb, bspo, wga, wgb, bgpo)
    return xs, xg


def kernel(surface_x, graph_x, rbf_weights,
           W_s_pre, b_s_pre, W_g_pre, b_g_pre,
           W_s_post, b_s_post, W_g_post, b_g_post):
    wsp = W_s_pre.astype(_BF16)
    wgp = W_g_pre.astype(_BF16)
    wsa = W_s_post[:D].astype(_BF16)
    wsb = W_s_post[D:].astype(_BF16)
    wga = W_g_post[:D].astype(_BF16)
    wgb = W_g_post[D:].astype(_BF16)
    bsp = b_s_pre.reshape(1, D)
    bgp = b_g_pre.reshape(1, D)
    bspo = b_s_post.reshape(1, D)
    bgpo = b_g_post.reshape(1, D)
    xs, xg = _run(surface_x, graph_x, rbf_weights, wsp, bsp, wgp, bgp,
                  wsa, wsb, bspo, wga, wgb, bgpo)
    return (xs, xg)
